# Initial kernel scaffold; baseline (speedup 1.0000x reference)
import jax
import jax.numpy as jnp
from jax.experimental import pallas as pl


def kernel(x, edge_index, emb_table, W1, b1, Wmu, bmu, Wlv, blv):
    raise NotImplementedError("write your pallas kernel here")


# speedup vs baseline: 19.8729x; 19.8729x over previous
import functools

import jax
import jax.numpy as jnp
from jax import lax
from jax.experimental import pallas as pl
from jax.experimental.pallas import tpu as pltpu
from jax.experimental.pallas import tpu_sc as plsc

N = 10000
E = 320000
D = 128
NC, NS = 2, 16
NW = NC * NS
EW = E // NW
K = 80
CH = EW // K
RPT = N // NS


def _sc_mesh():
    return plsc.VectorSubcoreMesh(
        core_axis_name="c", subcore_axis_name="s", num_cores=NC, num_subcores=NS
    )


@functools.cache
def _get_deg_kernel():
    return functools.partial(
        pl.kernel,
        out_type=jax.ShapeDtypeStruct((NC, NS, RPT, D), jnp.float32),
        mesh=_sc_mesh(),
        scratch_types=[
            pltpu.VMEM((CH, K), jnp.int32),
            pltpu.VMEM((K, D), jnp.float32),
            pltpu.VMEM_SHARED((N, D), jnp.float32),
        ],
    )(_deg_body)


def _deg_body(dst_hbm, ones_hbm, zero_hbm, out_hbm, dst_v, ones_v, acc_sh):
    c = lax.axis_index("c")
    s = lax.axis_index("s")
    wid = c * NS + s
    pltpu.sync_copy(dst_hbm.at[wid], dst_v)
    pltpu.sync_copy(ones_hbm.at[0, pl.ds(0, K)], ones_v)

    @pl.when(c == 0)
    def _():
        pltpu.sync_copy(ones_hbm.at[s], acc_sh.at[pl.ds(s * RPT, RPT)])

    @pl.when(c != 0)
    def _():
        pltpu.sync_copy(zero_hbm.at[s], acc_sh.at[pl.ds(s * RPT, RPT)])

    plsc.subcore_barrier()

    @pl.loop(0, CH)
    def _(i):
        pltpu.sync_copy(ones_v, acc_sh.at[dst_v.at[i]], add=True)

    plsc.subcore_barrier()
    pltpu.sync_copy(acc_sh.at[pl.ds(s * RPT, RPT)], out_hbm.at[c, s])


@functools.cache
def _get_msgpass_kernel():
    return functools.partial(
        pl.kernel,
        out_type=jax.ShapeDtypeStruct((NC, NS, RPT, D), jnp.float32),
        mesh=_sc_mesh(),
        scratch_types=[
            pltpu.VMEM((CH, K), jnp.int32),
            pltpu.VMEM((CH, K), jnp.int32),
            pltpu.VMEM((K, D), jnp.float32),
            pltpu.VMEM_SHARED((N, D), jnp.float32),
            pltpu.SemaphoreType.DMA,
        ],
    )(_msgpass_body)


def _msgpass_body(h_hbm, h_blk_hbm, src_hbm, dst_hbm, zero_hbm, out_hbm,
                  src_v, dst_v, rows_v, acc_sh, sem):
    c = lax.axis_index("c")
    s = lax.axis_index("s")
    wid = c * NS + s
    pltpu.sync_copy(src_hbm.at[wid], src_v)
    pltpu.sync_copy(dst_hbm.at[wid], dst_v)
    @pl.when(c == 0)
    def _():
        pltpu.sync_copy(h_blk_hbm.at[s], acc_sh.at[pl.ds(s * RPT, RPT)])

    @pl.when(c != 0)
    def _():
        pltpu.sync_copy(zero_hbm.at[s], acc_sh.at[pl.ds(s * RPT, RPT)])

    plsc.subcore_barrier()

    @pl.loop(0, CH)
    def _(i):
        pltpu.async_copy(h_hbm.at[src_v.at[i]], rows_v, sem).wait()
        pltpu.sync_copy(rows_v, acc_sh.at[dst_v.at[i]], add=True)

    plsc.subcore_barrier()
    pltpu.sync_copy(acc_sh.at[pl.ds(s * RPT, RPT)], out_hbm.at[c, s])


def _dinv_block(degp_ref):
    d = degp_ref[0, :, 0:1] + degp_ref[1, :, 0:1]
    return lax.rsqrt(d)


def _mm_scale_body(emb_ref, w_ref, degp_ref, out_ref):
    dinv = _dinv_block(degp_ref)
    h = jnp.dot(emb_ref[...], w_ref[...], preferred_element_type=jnp.float32)
    out_ref[...] = h * dinv


def _mid_body(s_ref, degp_ref, b_ref, w_ref, out_ref):
    dinv = _dinv_block(degp_ref)
    conv = dinv * (s_ref[0] + s_ref[1]) + b_ref[...]
    h = jnp.maximum(conv, 0.0)
    h2 = jnp.dot(h, w_ref[...], preferred_element_type=jnp.float32)
    out_ref[...] = h2 * dinv


def _final_body(s_ref, degp_ref, b_ref, out_ref):
    dinv = _dinv_block(degp_ref)
    out_ref[...] = dinv * (s_ref[0] + s_ref[1]) + b_ref[...]


RB = 1000
_GRID = N // RB


def _tc_call(body, in_specs, out_shape):
    return pl.pallas_call(
        body,
        grid=(_GRID,),
        in_specs=in_specs,
        out_specs=pl.BlockSpec((RB, D), lambda i: (i, 0)),
        out_shape=jax.ShapeDtypeStruct(out_shape, jnp.float32),
    )


_spec_rows = pl.BlockSpec((RB, D), lambda i: (i, 0))
_spec_w = pl.BlockSpec((D, D), lambda i: (0, 0))
_spec_degp = pl.BlockSpec((NC, RB, D), lambda i: (0, i, 0))
_spec_b = pl.BlockSpec((1, D), lambda i: (0, 0))
_spec_s = pl.BlockSpec((NC, RB, D), lambda i: (0, i, 0))


def kernel(x, edge_index, emb_table, W1, b1, Wmu, bmu, Wlv, blv):
    f32 = jnp.float32
    emb = jnp.take(emb_table, x, axis=0)
    src3 = edge_index[0].reshape(NW, CH, K)
    dst3 = edge_index[1].reshape(NW, CH, K)
    ones_d = jnp.ones((NS, RPT, D), f32)
    zero_d = jnp.zeros((NS, RPT, D), f32)
    W2 = jnp.concatenate([Wmu, Wlv], axis=1)
    b2 = jnp.concatenate([bmu, blv]).reshape(1, D)
    b1r = b1.reshape(1, D)

    degp = _get_deg_kernel()(dst3, ones_d, zero_d).reshape(NC, N, D)

    h1p = _tc_call(_mm_scale_body, [_spec_rows, _spec_w, _spec_degp], (N, D))(
        emb, W1, degp)

    s1 = _get_msgpass_kernel()(h1p, h1p.reshape(NS, RPT, D), src3, dst3, zero_d)
    s1 = s1.reshape(NC, N, D)

    h2p = _tc_call(_mid_body, [_spec_s, _spec_degp, _spec_b, _spec_w], (N, D))(
        s1, degp, b1r, W2)

    s2 = _get_msgpass_kernel()(h2p, h2p.reshape(NS, RPT, D), src3, dst3, zero_d)
    s2 = s2.reshape(NC, N, D)

    out = _tc_call(_final_body, [_spec_s, _spec_degp, _spec_b], (N, D))(
        s2, degp, b2)

    return (out[:, : D // 2], out[:, D // 2 :])


# speedup vs baseline: 25.2492x; 1.2705x over previous
import functools

import jax
import jax.numpy as jnp
from jax import lax
from jax.experimental import pallas as pl
from jax.experimental.pallas import tpu as pltpu
from jax.experimental.pallas import tpu_sc as plsc

N = 10000
E = 320000
D = 128
NC, NS = 2, 16
NW = NC * NS
EW = E // NW
K = 80
CH = EW // K
RPT = N // NS


def _sc_mesh():
    return plsc.VectorSubcoreMesh(
        core_axis_name="c", subcore_axis_name="s", num_cores=NC, num_subcores=NS
    )


OUT_OFF = 624
OUT_WIN = 640
XCH = 125
XNJ = RPT // XCH


@functools.cache
def _get_deg_kernel():
    return functools.partial(
        pl.kernel,
        out_type=(
            jax.ShapeDtypeStruct((NC * N,), jnp.float32),
            jax.ShapeDtypeStruct((NS, XNJ, XCH, D), jnp.float32),
        ),
        mesh=_sc_mesh(),
        scratch_types=[
            pltpu.VMEM((CH, K), jnp.int32),
            pltpu.VMEM((K,), jnp.float32),
            pltpu.VMEM((XNJ, XCH), jnp.int32),
            pltpu.VMEM((XCH, D), jnp.float32),
            pltpu.VMEM((OUT_WIN,), jnp.float32),
            pltpu.VMEM_SHARED((N,), jnp.float32),
            pltpu.SemaphoreType.DMA,
        ],
    )(_deg_body)


def _deg_body(dst_hbm, x_hbm, emb_hbm, ones_hbm, deg_out, emb_out,
              dst_v, ones_v, x_v, erows_v, stage_v, acc_sh, sem):
    c = lax.axis_index("c")
    s = lax.axis_index("s")
    wid = c * NS + s
    pltpu.sync_copy(dst_hbm.at[wid], dst_v)
    pltpu.sync_copy(ones_hbm, ones_v)
    for j in range(OUT_WIN // 16):
        stage_v[pl.ds(16 * j, 16)] = jnp.zeros((16,), jnp.float32)
    pltpu.sync_copy(stage_v, acc_sh.at[pl.ds(s * OUT_OFF, OUT_WIN)])

    @pl.when(c == 0)
    def _():
        pltpu.sync_copy(x_hbm.at[s], x_v)

    plsc.subcore_barrier()

    @pl.loop(0, CH)
    def _(i):
        pltpu.sync_copy(ones_v, acc_sh.at[dst_v.at[i]], add=True)

    @pl.when(c == 0)
    def _():
        for j in range(XNJ):
            pltpu.async_copy(emb_hbm.at[x_v.at[j]], erows_v, sem).wait()
            pltpu.sync_copy(erows_v, emb_out.at[s, j])

    plsc.subcore_barrier()
    pltpu.sync_copy(acc_sh.at[pl.ds(s * OUT_OFF, OUT_WIN)], stage_v)
    pltpu.sync_copy(stage_v, deg_out.at[pl.ds(c * N + s * OUT_OFF, OUT_WIN)])


@functools.cache
def _get_msgpass_kernel():
    return functools.partial(
        pl.kernel,
        out_type=jax.ShapeDtypeStruct((NC, NS, RPT, D), jnp.float32),
        mesh=_sc_mesh(),
        scratch_types=[
            pltpu.VMEM((EW,), jnp.int32),
            pltpu.VMEM((CH, K), jnp.int32),
            pltpu.VMEM((K, D), jnp.float32),
            pltpu.VMEM((K, D), jnp.float32),
            pltpu.VMEM_SHARED((N, D), jnp.float32),
            pltpu.SemaphoreType.DMA,
            pltpu.SemaphoreType.DMA,
        ],
    )(_msgpass_body)


def _msgpass_body(h_hbm, h_blk_hbm, src_hbm, dst_hbm, zero_hbm, out_hbm,
                  src_v, dst_v, rows_a, rows_b, acc_sh, sem_a, sem_b):
    c = lax.axis_index("c")
    s = lax.axis_index("s")
    wid = c * NS + s
    pltpu.sync_copy(src_hbm.at[pl.ds(wid * EW, EW)], src_v)
    pltpu.sync_copy(dst_hbm.at[wid], dst_v)
    @pl.when(c == 0)
    def _():
        pltpu.sync_copy(h_blk_hbm.at[s], acc_sh.at[pl.ds(s * RPT, RPT)])

    @pl.when(c != 0)
    def _():
        pltpu.sync_copy(zero_hbm.at[s], acc_sh.at[pl.ds(s * RPT, RPT)])

    plsc.subcore_barrier()

    @pl.loop(0, CH // 2)
    def _(t):
        i0 = 2 * t
        sa = src_v.at[pl.ds(i0 * K, K)]
        sb = src_v.at[pl.ds((i0 + 1) * K, K)]
        pltpu.async_copy(h_hbm.at[sa], rows_a, sem_a)
        pltpu.async_copy(h_hbm.at[sb], rows_b, sem_b)
        pltpu.make_async_copy(h_hbm.at[sa], rows_a, sem_a).wait()
        pltpu.sync_copy(rows_a, acc_sh.at[dst_v.at[i0]], add=True)
        pltpu.make_async_copy(h_hbm.at[sb], rows_b, sem_b).wait()
        pltpu.sync_copy(rows_b, acc_sh.at[dst_v.at[i0 + 1]], add=True)

    sl = src_v.at[pl.ds((CH - 1) * K, K)]
    pltpu.async_copy(h_hbm.at[sl], rows_a, sem_a).wait()
    pltpu.sync_copy(rows_a, acc_sh.at[dst_v.at[CH - 1]], add=True)

    plsc.subcore_barrier()
    pltpu.sync_copy(acc_sh.at[pl.ds(s * RPT, RPT)], out_hbm.at[c, s])


def _dinv_block(d0_ref, d1_ref):
    d = 1.0 + d0_ref[0] + d1_ref[0]
    return lax.rsqrt(d)


def _mm_scale_body(emb_ref, w_ref, d0_ref, d1_ref, out_ref):
    dinv = _dinv_block(d0_ref, d1_ref)
    h = jnp.dot(emb_ref[...], w_ref[...], preferred_element_type=jnp.float32)
    out_ref[...] = h * dinv


def _mid_body(s_ref, d0_ref, d1_ref, b_ref, w_ref, out_ref):
    dinv = _dinv_block(d0_ref, d1_ref)
    conv = dinv * (s_ref[0] + s_ref[1]) + b_ref[...]
    h = jnp.maximum(conv, 0.0)
    h2 = jnp.dot(h, w_ref[...], preferred_element_type=jnp.float32)
    out_ref[...] = h2 * dinv


def _final_body(s_ref, d0_ref, d1_ref, b_ref, out_ref):
    dinv = _dinv_block(d0_ref, d1_ref)
    out_ref[...] = dinv * (s_ref[0] + s_ref[1]) + b_ref[...]


RB = 1000
_GRID = N // RB


def _tc_call(body, in_specs, out_shape):
    return pl.pallas_call(
        body,
        grid=(_GRID,),
        in_specs=in_specs,
        out_specs=pl.BlockSpec((RB, D), lambda i: (i, 0)),
        out_shape=jax.ShapeDtypeStruct(out_shape, jnp.float32),
    )


_spec_rows = pl.BlockSpec((RB, D), lambda i: (i, 0))
_spec_w = pl.BlockSpec((D, D), lambda i: (0, 0))
_spec_degc = pl.BlockSpec((1, RB, 1), lambda i: (i, 0, 0))
_spec_b = pl.BlockSpec((1, D), lambda i: (0, 0))
_spec_s = pl.BlockSpec((NC, RB, D), lambda i: (0, i, 0))


def kernel(x, edge_index, emb_table, W1, b1, Wmu, bmu, Wlv, blv):
    f32 = jnp.float32
    src1 = edge_index[0]
    dst3 = edge_index[1].reshape(NW, CH, K)
    x5 = x.reshape(NS, XNJ, XCH)
    ones1 = jnp.ones((K,), f32)
    zero_d = jnp.zeros((NS, RPT, D), f32)
    W2 = jnp.concatenate([Wmu, Wlv], axis=1)
    b2 = jnp.concatenate([bmu, blv]).reshape(1, D)
    b1r = b1.reshape(1, D)

    deg1d, emb4 = _get_deg_kernel()(dst3, x5, emb_table, ones1)
    d0 = deg1d[:N].reshape(_GRID, RB, 1)
    d1 = deg1d[N:].reshape(_GRID, RB, 1)
    emb = emb4.reshape(N, D)

    h1p = _tc_call(
        _mm_scale_body, [_spec_rows, _spec_w, _spec_degc, _spec_degc], (N, D)
    )(emb, W1, d0, d1)

    s1 = _get_msgpass_kernel()(h1p, h1p.reshape(NS, RPT, D), src1, dst3, zero_d)
    s1 = s1.reshape(NC, N, D)

    h2p = _tc_call(
        _mid_body, [_spec_s, _spec_degc, _spec_degc, _spec_b, _spec_w], (N, D)
    )(s1, d0, d1, b1r, W2)

    s2 = _get_msgpass_kernel()(h2p, h2p.reshape(NS, RPT, D), src1, dst3, zero_d)
    s2 = s2.reshape(NC, N, D)

    out = _tc_call(
        _final_body, [_spec_s, _spec_degc, _spec_degc, _spec_b], (N, D)
    )(s2, d0, d1, b2)

    return (out[:, : D // 2], out[:, D // 2 :])


# speedup vs baseline: 30.5779x; 1.2110x over previous
import functools

import jax
import jax.numpy as jnp
from jax import lax
from jax.experimental import pallas as pl
from jax.experimental.pallas import tpu as pltpu
from jax.experimental.pallas import tpu_sc as plsc

N = 10000
E = 320000
D = 128
NC, NS = 2, 16
NW = NC * NS
EW = E // NW
K = 80
CH = EW // K
RPT = N // NS


def _sc_mesh():
    return plsc.VectorSubcoreMesh(
        core_axis_name="c", subcore_axis_name="s", num_cores=NC, num_subcores=NS
    )


OUT_OFF = 624
OUT_WIN = 640
XCH = 125
XNJ = RPT // XCH


@functools.cache
def _get_deg_kernel():
    return functools.partial(
        pl.kernel,
        out_type=(
            jax.ShapeDtypeStruct((NC * N,), jnp.float32),
            jax.ShapeDtypeStruct((NS, XNJ, XCH, D), jnp.float32),
        ),
        mesh=_sc_mesh(),
        scratch_types=[
            pltpu.VMEM((CH, K), jnp.int32),
            pltpu.VMEM((K,), jnp.float32),
            pltpu.VMEM((XNJ, XCH), jnp.int32),
            pltpu.VMEM((XCH, D), jnp.float32),
            pltpu.VMEM((OUT_WIN,), jnp.float32),
            pltpu.VMEM_SHARED((N,), jnp.float32),
            pltpu.SemaphoreType.DMA,
        ],
    )(_deg_body)


def _deg_body(dst_hbm, x_hbm, emb_hbm, ones_hbm, deg_out, emb_out,
              dst_v, ones_v, x_v, erows_v, stage_v, acc_sh, sem):
    c = lax.axis_index("c")
    s = lax.axis_index("s")
    wid = c * NS + s
    pltpu.sync_copy(dst_hbm.at[wid], dst_v)
    pltpu.sync_copy(ones_hbm, ones_v)
    for j in range(OUT_WIN // 16):
        stage_v[pl.ds(16 * j, 16)] = jnp.zeros((16,), jnp.float32)
    pltpu.sync_copy(stage_v, acc_sh.at[pl.ds(s * OUT_OFF, OUT_WIN)])

    @pl.when(c == 0)
    def _():
        pltpu.sync_copy(x_hbm.at[s], x_v)

    plsc.subcore_barrier()

    @pl.loop(0, CH)
    def _(i):
        pltpu.sync_copy(ones_v, acc_sh.at[dst_v.at[i]], add=True)

    @pl.when(c == 0)
    def _():
        for j in range(XNJ):
            pltpu.async_copy(emb_hbm.at[x_v.at[j]], erows_v, sem).wait()
            pltpu.sync_copy(erows_v, emb_out.at[s, j])

    plsc.subcore_barrier()
    pltpu.sync_copy(acc_sh.at[pl.ds(s * OUT_OFF, OUT_WIN)], stage_v)
    pltpu.sync_copy(stage_v, deg_out.at[pl.ds(c * N + s * OUT_OFF, OUT_WIN)])


@functools.cache
def _get_msgpass_kernel():
    return functools.partial(
        pl.kernel,
        out_type=jax.ShapeDtypeStruct((NC, NS, RPT, D), jnp.float32),
        mesh=_sc_mesh(),
        scratch_types=[
            pltpu.VMEM((EW,), jnp.int32),
            pltpu.VMEM((CH, K), jnp.int32),
            pltpu.VMEM((K, D), jnp.float32),
            pltpu.VMEM((K, D), jnp.float32),
            pltpu.VMEM_SHARED((N, D), jnp.float32),
            pltpu.SemaphoreType.DMA,
            pltpu.SemaphoreType.DMA,
        ],
    )(_msgpass_body)


def _msgpass_body(h_hbm, h_blk_hbm, src_hbm, dst_hbm, zero_hbm, out_hbm,
                  src_v, dst_v, rows_a, rows_b, acc_sh, sem_a, sem_b):
    c = lax.axis_index("c")
    s = lax.axis_index("s")
    wid = c * NS + s
    pltpu.sync_copy(src_hbm.at[pl.ds(wid * EW, EW)], src_v)
    pltpu.sync_copy(dst_hbm.at[wid], dst_v)
    @pl.when(c == 0)
    def _():
        pltpu.sync_copy(h_blk_hbm.at[s], acc_sh.at[pl.ds(s * RPT, RPT)])

    @pl.when(c != 0)
    def _():
        pltpu.sync_copy(zero_hbm.at[s], acc_sh.at[pl.ds(s * RPT, RPT)])

    plsc.subcore_barrier()

    def _gather(i, buf, sem):
        pltpu.async_copy(h_hbm.at[src_v.at[pl.ds(i * K, K)]], buf, sem)

    def _gwait(i, buf, sem):
        pltpu.make_async_copy(
            h_hbm.at[src_v.at[pl.ds(i * K, K)]], buf, sem).wait()

    _gather(0, rows_a, sem_a)
    _gather(1, rows_b, sem_b)

    @pl.loop(0, CH // 2)
    def _(t):
        i0 = 2 * t
        _gwait(i0, rows_a, sem_a)
        pltpu.sync_copy(rows_a, acc_sh.at[dst_v.at[i0]], add=True)
        _gather(i0 + 2, rows_a, sem_a)
        _gwait(i0 + 1, rows_b, sem_b)
        pltpu.sync_copy(rows_b, acc_sh.at[dst_v.at[i0 + 1]], add=True)

        @pl.when(t < CH // 2 - 1)
        def _():
            _gather(i0 + 3, rows_b, sem_b)

    _gwait(CH - 1, rows_a, sem_a)
    pltpu.sync_copy(rows_a, acc_sh.at[dst_v.at[CH - 1]], add=True)

    plsc.subcore_barrier()
    pltpu.sync_copy(acc_sh.at[pl.ds(s * RPT, RPT)], out_hbm.at[c, s])


def _dinv_block(d0_ref, d1_ref):
    d = 1.0 + d0_ref[0] + d1_ref[0]
    return lax.rsqrt(d)


def _mm_scale_body(emb_ref, w_ref, d0_ref, d1_ref, out_ref):
    dinv = _dinv_block(d0_ref, d1_ref)
    h = jnp.dot(emb_ref[...], w_ref[...], preferred_element_type=jnp.float32)
    out_ref[...] = h * dinv


def _mid_body(s_ref, d0_ref, d1_ref, b_ref, w_ref, out_ref):
    dinv = _dinv_block(d0_ref, d1_ref)
    conv = dinv * (s_ref[0] + s_ref[1]) + b_ref[...]
    h = jnp.maximum(conv, 0.0)
    h2 = jnp.dot(h, w_ref[...], preferred_element_type=jnp.float32)
    out_ref[...] = h2 * dinv


def _final_body(s_ref, d0_ref, d1_ref, b_ref, out_ref):
    dinv = _dinv_block(d0_ref, d1_ref)
    out_ref[...] = dinv * (s_ref[0] + s_ref[1]) + b_ref[...]


RB = 1000
_GRID = N // RB


def _tc_call(body, in_specs, out_shape):
    return pl.pallas_call(
        body,
        grid=(_GRID,),
        in_specs=in_specs,
        out_specs=pl.BlockSpec((RB, D), lambda i: (i, 0)),
        out_shape=jax.ShapeDtypeStruct(out_shape, jnp.float32),
    )


_spec_rows = pl.BlockSpec((RB, D), lambda i: (i, 0))
_spec_w = pl.BlockSpec((D, D), lambda i: (0, 0))
_spec_degc = pl.BlockSpec((1, RB, 1), lambda i: (i, 0, 0))
_spec_b = pl.BlockSpec((1, D), lambda i: (0, 0))
_spec_s = pl.BlockSpec((NC, RB, D), lambda i: (0, i, 0))


def kernel(x, edge_index, emb_table, W1, b1, Wmu, bmu, Wlv, blv):
    f32 = jnp.float32
    src1 = edge_index[0]
    dst3 = edge_index[1].reshape(NW, CH, K)
    x5 = x.reshape(NS, XNJ, XCH)
    ones1 = jnp.ones((K,), f32)
    zero_d = jnp.zeros((NS, RPT, D), f32)
    W2 = jnp.concatenate([Wmu, Wlv], axis=1)
    b2 = jnp.concatenate([bmu, blv]).reshape(1, D)
    b1r = b1.reshape(1, D)

    deg1d, emb4 = _get_deg_kernel()(dst3, x5, emb_table, ones1)
    d0 = deg1d[:N].reshape(_GRID, RB, 1)
    d1 = deg1d[N:].reshape(_GRID, RB, 1)
    emb = emb4.reshape(N, D)

    h1p = _tc_call(
        _mm_scale_body, [_spec_rows, _spec_w, _spec_degc, _spec_degc], (N, D)
    )(emb, W1, d0, d1)

    s1 = _get_msgpass_kernel()(h1p, h1p.reshape(NS, RPT, D), src1, dst3, zero_d)
    s1 = s1.reshape(NC, N, D)

    h2p = _tc_call(
        _mid_body, [_spec_s, _spec_degc, _spec_degc, _spec_b, _spec_w], (N, D)
    )(s1, d0, d1, b1r, W2)

    s2 = _get_msgpass_kernel()(h2p, h2p.reshape(NS, RPT, D), src1, dst3, zero_d)
    s2 = s2.reshape(NC, N, D)

    out = _tc_call(
        _final_body, [_spec_s, _spec_degc, _spec_degc, _spec_b], (N, D)
    )(s2, d0, d1, b2)

    return (out[:, : D // 2], out[:, D // 2 :])
